# baseline (speedup 1.0000x reference)
import functools

import jax
import jax.numpy as jnp
from jax import lax
from jax.experimental import pallas as pl
from jax.experimental.pallas import tpu as pltpu
from jax.experimental.pallas import tpu_sc as plsc

D = 384
DWR = D // 2
DW = 256
N_NODE = 128
N_DEPTH = 32
N_TOK = 4 * 8192

NC = 2
NS = 16
L = 16
NW = NC * NS
TOK_W = N_TOK // NW
CH = 64
NCH = TOK_W // CH
NB = 3
NF = 2


def _prep_body(node_ref, depth_ref, nid_ref, did_ref, ctab_ref, cidx_ref):
    node = node_ref[...]
    depth = depth_ref[...]
    ctab_ref[...] = node[:, None, :] + depth[None, :, :]
    cidx_ref[...] = nid_ref[...] * N_DEPTH + did_ref[...]


def _prep(node_table, depth_table, nid, did):
    ctab, cidx = pl.pallas_call(
        _prep_body,
        out_shape=(
            jax.ShapeDtypeStruct((N_NODE, N_DEPTH, D), jnp.float32),
            jax.ShapeDtypeStruct(nid.shape, jnp.int32),
        ),
    )(node_table, depth_table, nid, did)
    cs = ctab.reshape(N_NODE * N_DEPTH, D // 32, 2, 16).swapaxes(-1, -2)
    cw = jax.lax.bitcast_convert_type(cs.astype(jnp.bfloat16), jnp.int32)
    cw = cw.reshape(N_NODE * N_DEPTH, DWR)
    cw = jnp.pad(cw, ((0, 0), (0, DW - DWR)))
    return cw, cidx.reshape(-1)


def _sc_body(cidx_hbm, ctab_hbm, out_hbm, cidx_v, rw_v, rf_v, gsem, ssem):
    wid = lax.axis_index("s") * NC + lax.axis_index("c")
    base = wid * TOK_W
    pltpu.sync_copy(cidx_hbm.at[pl.ds(base, TOK_W)], cidx_v)

    def _gather(c):
        idx = cidx_v.at[pl.ds(c * CH, CH)]
        return pltpu.async_copy(ctab_hbm.at[idx], rw_v.at[c % NB], gsem)

    def _convert(c):
        src = rw_v.at[c % NB]
        dst = rf_v.at[c % NF]
        mask = jnp.full((L,), jnp.int32(-65536))

        def _row(r, carry):
            for j in range(D // 32):
                w = src[r, pl.ds(j * L, L)]
                lo = plsc.bitcast(w << 16, jnp.float32)
                hi = plsc.bitcast(w & mask, jnp.float32)
                dst[r, pl.ds(j * 32, L)] = lo
                dst[r, pl.ds(j * 32 + L, L)] = hi
            return carry

        lax.fori_loop(0, CH, _row, 0)

    def _scatter(c):
        return pltpu.async_copy(
            rf_v.at[c % NF], out_hbm.at[pl.ds(base + c * CH, CH)], ssem)

    gathers = [None] * NCH
    scatters = [None] * NCH
    for c in range(NB - 1):
        gathers[c] = _gather(c)
    for c in range(NCH):
        gathers[c].wait()
        nxt = c + NB - 1
        if nxt < NCH:
            gathers[nxt] = _gather(nxt)
        if c - NF >= 0:
            scatters[c - NF].wait()
        _convert(c)
        scatters[c] = _scatter(c)
    scatters[NCH - 2].wait()
    scatters[NCH - 1].wait()


@jax.jit
def _run(node_ids, depth_ids, node_table, depth_table):
    ctab, cidx = _prep(node_table, depth_table, node_ids, depth_ids)
    k = functools.partial(
        pl.kernel,
        out_type=jax.ShapeDtypeStruct((N_TOK, D), jnp.float32),
        mesh=plsc.VectorSubcoreMesh(core_axis_name="c", subcore_axis_name="s"),
        compiler_params=pltpu.CompilerParams(needs_layout_passes=False),
        scratch_types=[
            pltpu.VMEM((TOK_W,), jnp.int32),
            pltpu.VMEM((NB, CH, DW), jnp.int32),
            pltpu.VMEM((NF, CH, D), jnp.float32),
            pltpu.SemaphoreType.DMA,
            pltpu.SemaphoreType.DMA,
        ],
    )(_sc_body)
    return k(cidx, ctab)


def kernel(node_type_ids, depth_ids, node_table, depth_table):
    b, t = node_type_ids.shape
    nid = node_type_ids.astype(jnp.int32)
    did = depth_ids.astype(jnp.int32)
    out = _run(nid, did, node_table, depth_table)
    return out.reshape(b, t, D)


# speedup vs baseline: 2.2305x; 2.2305x over previous
import functools

import jax
import jax.numpy as jnp
from jax import lax
from jax.experimental import pallas as pl
from jax.experimental.pallas import tpu as pltpu
from jax.experimental.pallas import tpu_sc as plsc

D = 384
N_NODE = 128
N_DEPTH = 32
N_TOK = 4 * 8192

NC = 2
NS = 16
L = 16
NW = NC * NS
TOK_W = N_TOK // NW
CH = 64
NCH = TOK_W // CH
NB = 5


def _prep_body(node_ref, depth_ref, nid_ref, did_ref, ctab_ref, cidx_ref):
    node = node_ref[...]
    depth = depth_ref[...]
    ctab_ref[...] = node[:, None, :] + depth[None, :, :]
    cidx_ref[...] = nid_ref[...] * N_DEPTH + did_ref[...]


def _prep(node_table, depth_table, nid, did):
    ctab, cidx = pl.pallas_call(
        _prep_body,
        out_shape=(
            jax.ShapeDtypeStruct((N_NODE, N_DEPTH, D), jnp.float32),
            jax.ShapeDtypeStruct(nid.shape, jnp.int32),
        ),
    )(node_table, depth_table, nid, did)
    return ctab.reshape(N_NODE * N_DEPTH, D), cidx.reshape(-1)


def _sc_body(cidx_hbm, ctab_hbm, out_hbm, cidx_v, rows_v, gsem, ssem):
    wid = lax.axis_index("s") * NC + lax.axis_index("c")
    base = wid * TOK_W
    pltpu.sync_copy(cidx_hbm.at[pl.ds(base, TOK_W)], cidx_v)

    def _gather(c):
        idx = cidx_v.at[pl.ds(c * CH, CH)]
        return pltpu.async_copy(ctab_hbm.at[idx], rows_v.at[c % NB], gsem)

    def _scatter(c):
        return pltpu.async_copy(
            rows_v.at[c % NB], out_hbm.at[pl.ds(base + c * CH, CH)], ssem)

    gathers = [None] * NCH
    scatters = [None] * NCH
    for c in range(NB - 1):
        gathers[c] = _gather(c)
    for c in range(NCH):
        gathers[c].wait()
        nxt = c + NB - 1
        if nxt < NCH:
            if c - 1 >= 0:
                scatters[c - 1].wait()
            gathers[nxt] = _gather(nxt)
        scatters[c] = _scatter(c)
    for c in range(NCH - NB, NCH):
        if c >= 0:
            scatters[c].wait()


@jax.jit
def _run(node_ids, depth_ids, node_table, depth_table):
    ctab, cidx = _prep(node_table, depth_table, node_ids, depth_ids)
    k = functools.partial(
        pl.kernel,
        out_type=jax.ShapeDtypeStruct((N_TOK, D), jnp.float32),
        mesh=plsc.VectorSubcoreMesh(core_axis_name="c", subcore_axis_name="s"),
        scratch_types=[
            pltpu.VMEM((TOK_W,), jnp.int32),
            pltpu.VMEM((NB, CH, D), jnp.float32),
            pltpu.SemaphoreType.DMA,
            pltpu.SemaphoreType.DMA,
        ],
    )(_sc_body)
    return k(cidx, ctab)


def kernel(node_type_ids, depth_ids, node_table, depth_table):
    b, t = node_type_ids.shape
    nid = node_type_ids.astype(jnp.int32)
    did = depth_ids.astype(jnp.int32)
    out = _run(nid, did, node_table, depth_table)
    return out.reshape(b, t, D)
